# Initial kernel scaffold; baseline (speedup 1.0000x reference)
import jax
import jax.numpy as jnp
from jax.experimental import pallas as pl


def kernel(x, edge_index, params):
    raise NotImplementedError("write your pallas kernel here")


# speedup vs baseline: 2.9884x; 2.9884x over previous
import functools

import jax
import jax.numpy as jnp
from jax import lax
from jax.experimental import pallas as pl
from jax.experimental.pallas import tpu as pltpu
from jax.experimental.pallas import tpu_sc as plsc

F32 = jnp.float32

_N = 10000
_D = 128
_HF = 64
_NC = 2
_NS = 16
_NW = _NC * _NS
_CH = 128
_NPAD = 10240
_ZR = 64
_OB = 128
_BN = 1000
_G = _N // _BN


def _make_spmv(K):
    mesh = plsc.VectorSubcoreMesh(core_axis_name="c", subcore_axis_name="s",
                                  num_cores=_NC, num_subcores=_NS)

    @functools.partial(
        pl.kernel,
        out_type=jax.ShapeDtypeStruct((_NC, 2, _NPAD, _HF), F32),
        mesh=mesh,
        scratch_types=[
            pltpu.VMEM((K, _CH), jnp.int32),
            pltpu.VMEM((K, _CH), jnp.int32),
            pltpu.VMEM((_CH, _HF), F32),
            pltpu.VMEM((_ZR, _HF), F32),
            pltpu.VMEM((_OB, _HF), F32),
            pltpu.VMEM_SHARED((_NPAD, _HF), F32),
            pltpu.SemaphoreType.DMA,
        ],
        compiler_params=pltpu.CompilerParams(use_tc_tiling_on_sc=False),
    )
    def spmv(u_hbm, src_hbm, dst_hbm, z_hbm, out_hbm,
             src_v, dst_v, rows_v, zer_v, ob_v, agg_s, sem):
        c = lax.axis_index("c")
        s = lax.axis_index("s")
        wid = s * _NC + c

        pltpu.sync_copy(src_hbm.at[wid], src_v)
        pltpu.sync_copy(dst_hbm.at[wid], dst_v)
        pltpu.sync_copy(z_hbm, zer_v)

        rows_per_sub = _NPAD // _NS

        for f in range(2):
            def zdma(i, carry):
                pltpu.sync_copy(zer_v,
                                agg_s.at[pl.ds(s * rows_per_sub + i * _ZR, _ZR)])
                return carry

            lax.fori_loop(0, rows_per_sub // _ZR, zdma, 0)
            plsc.subcore_barrier()

            def chunk(k, carry):
                pltpu.async_copy(u_hbm.at[f].at[src_v.at[k]], rows_v, sem).wait()
                pltpu.sync_copy(rows_v, agg_s.at[dst_v.at[k]], add=True)
                return carry

            lax.fori_loop(0, K, chunk, 0)
            plsc.subcore_barrier()

            def odma(i, carry):
                base = s * rows_per_sub + i * _OB
                pltpu.sync_copy(agg_s.at[pl.ds(base, _OB)], ob_v)
                pltpu.sync_copy(ob_v, out_hbm.at[c].at[f].at[pl.ds(base, _OB)])
                return carry

            lax.fori_loop(0, rows_per_sub // _OB, odma, 0)
            plsc.subcore_barrier()

    return spmv


def _mmT(a, w):
    return lax.dot_general(a, w, (((1,), (1,)), ((), ())),
                           preferred_element_type=F32)


def _split_store(t, o_ref):
    o_ref[0] = t[:, 0:_HF]
    o_ref[1] = t[:, _HF:_D]


def _assemble(p_ref):
    return jnp.concatenate([p_ref[0, 0] + p_ref[1, 0],
                            p_ref[0, 1] + p_ref[1, 1]], axis=1)


def _h128(h_ref):
    return jnp.concatenate([h_ref[0], h_ref[1]], axis=1)


def _cnt(c_ref):
    return jnp.maximum(c_ref[0, 0][:, 0:1] + c_ref[1, 0][:, 0:1], 1.0)


def _split_body(x_ref, o_ref):
    _split_store(x_ref[...], o_ref)


def _conv1_body(p_ref, c_ref, h_ref, wl_ref, bl_ref, wr_ref, o_ref):
    mean = _assemble(p_ref) / _cnt(c_ref)
    pre = _mmT(mean, wl_ref[...]) + bl_ref[...] + _mmT(_h128(h_ref), wr_ref[...])
    _split_store(jnp.maximum(pre, 0.0), o_ref)


def _conv2_body(p_ref, c_ref, h_ref, wl_ref, bl_ref, wr_ref, o_ref, st_ref):
    i = pl.program_id(0)
    mean = _assemble(p_ref) / _cnt(c_ref)
    h2 = _mmT(mean, wl_ref[...]) + bl_ref[...] + _mmT(_h128(h_ref), wr_ref[...])
    _split_store(h2, o_ref)

    @pl.when(i == 0)
    def _():
        st_ref[...] = jnp.zeros((8, _D), F32)

    st_ref[0:1, :] = st_ref[0:1, :] + jnp.sum(h2, axis=0, keepdims=True)
    st_ref[1:2, :] = st_ref[1:2, :] + jnp.sum(h2 * h2, axis=0, keepdims=True)


def _bn_relu(h2, st_ref, g_ref, b_ref):
    mu = st_ref[0:1, :] * (1.0 / _N)
    var = st_ref[1:2, :] * (1.0 / _N) - mu * mu
    return jnp.maximum((h2 - mu) / jnp.sqrt(var + 1e-5) * g_ref[...] + b_ref[...],
                       0.0)


def _bn_body(h2_ref, st_ref, g_ref, b_ref, o_ref):
    _split_store(_bn_relu(_h128(h2_ref), st_ref, g_ref, b_ref), o_ref)


def _final_body(h2_ref, st_ref, g_ref, b_ref, w0_ref, b0_ref, w1_ref, b1_ref,
                w2_ref, b2_ref, o_ref, sacc, macc):
    i = pl.program_id(0)
    hn = _bn_relu(_h128(h2_ref), st_ref, g_ref, b_ref)

    @pl.when(i == 0)
    def _():
        sacc[...] = jnp.zeros((1, _D), F32)
        macc[...] = jnp.full((1, _D), -jnp.inf, F32)

    sacc[...] = sacc[...] + jnp.sum(hn, axis=0, keepdims=True)
    macc[...] = jnp.maximum(macc[...], jnp.max(hn, axis=0, keepdims=True))

    @pl.when(i == _G - 1)
    def _():
        mean = sacc[...] * (1.0 / _N)
        mx = macc[...]
        g1 = jnp.maximum(_mmT(mean, w0_ref[:, 0:_D]) + _mmT(mx, w0_ref[:, _D:2 * _D])
                         + b0_ref[...], 0.0)
        g2 = jnp.maximum(_mmT(g1, w1_ref[...]) + b1_ref[...], 0.0)
        o_ref[...] = _mmT(g2, w2_ref[...]) + b2_ref[...]


def _blk(shape, imap):
    return pl.BlockSpec(shape, imap)


_ROW = lambda i: (i, 0)
_FIX = lambda i: (0, 0)
_P3 = lambda i: (0, i, 0)
_P4 = lambda i: (0, 0, i, 0)

_PBLK = (_NC, 2, _BN, _HF)
_HBLK = (2, _BN, _HF)
_HSHAPE = jax.ShapeDtypeStruct((2, _N, _HF), F32)


def _split(x):
    return pl.pallas_call(
        _split_body, grid=(_G,),
        in_specs=[_blk((_BN, _D), _ROW)],
        out_specs=_blk(_HBLK, _P3),
        out_shape=_HSHAPE,
    )(x)


def _conv1(p, cparts, h, wl, bl, wr):
    return pl.pallas_call(
        _conv1_body, grid=(_G,),
        in_specs=[_blk(_PBLK, _P4), _blk(_PBLK, _P4), _blk(_HBLK, _P3),
                  _blk((_D, _D), _FIX), _blk((1, _D), _FIX), _blk((_D, _D), _FIX)],
        out_specs=_blk(_HBLK, _P3),
        out_shape=_HSHAPE,
    )(p, cparts, h, wl, bl, wr)


def _conv2(p, cparts, h, wl, bl, wr):
    return pl.pallas_call(
        _conv2_body, grid=(_G,),
        in_specs=[_blk(_PBLK, _P4), _blk(_PBLK, _P4), _blk(_HBLK, _P3),
                  _blk((_D, _D), _FIX), _blk((1, _D), _FIX), _blk((_D, _D), _FIX)],
        out_specs=[_blk(_HBLK, _P3), _blk((8, _D), _FIX)],
        out_shape=[_HSHAPE, jax.ShapeDtypeStruct((8, _D), F32)],
    )(p, cparts, h, wl, bl, wr)


def _bn(h2, st, g, b):
    return pl.pallas_call(
        _bn_body, grid=(_G,),
        in_specs=[_blk(_HBLK, _P3), _blk((8, _D), _FIX),
                  _blk((1, _D), _FIX), _blk((1, _D), _FIX)],
        out_specs=_blk(_HBLK, _P3),
        out_shape=_HSHAPE,
    )(h2, st, g, b)


def _final(h2, st, g, b, w0, b0, w1, b1, w2, b2):
    return pl.pallas_call(
        _final_body, grid=(_G,),
        in_specs=[_blk(_HBLK, _P3), _blk((8, _D), _FIX),
                  _blk((1, _D), _FIX), _blk((1, _D), _FIX),
                  _blk((_D, 2 * _D), _FIX), _blk((1, _D), _FIX),
                  _blk((_D // 2, _D), _FIX), _blk((1, _D // 2), _FIX),
                  _blk((_D, _D // 2), _FIX), _blk((1, _D), _FIX)],
        out_specs=_blk((1, _D), _FIX),
        out_shape=jax.ShapeDtypeStruct((1, _D), F32),
        scratch_shapes=[pltpu.VMEM((1, _D), F32), pltpu.VMEM((1, _D), F32)],
    )(h2, st, g, b, w0, b0, w1, b1, w2, b2)


def kernel(x, edge_index, params):
    convs, bns, cls = params["convs"], params["bns"], params["cls"]

    src, dst = edge_index[0], edge_index[1]
    e = src.shape[0]
    k = -(-e // (_NW * _CH))
    pad = k * _NW * _CH - e
    src_r = jnp.concatenate([src, jnp.zeros((pad,), jnp.int32)]).reshape(_NW, k, _CH)
    dst_r = jnp.concatenate([dst, jnp.full((pad,), _N, jnp.int32)]).reshape(_NW, k, _CH)
    zrows = jnp.zeros((_ZR, _HF), F32)

    spmv = _make_spmv(k)
    cparts = spmv(jnp.ones((2, _N, _HF), F32), src_r, dst_r, zrows)

    h = _split(x)
    out = None
    for i in range(4):
        l1, l2, bn = convs[i]["l1"], convs[i]["l2"], bns[i]
        p1 = spmv(h, src_r, dst_r, zrows)
        h = _conv1(p1, cparts, h, l1["Wl"], l1["bl"].reshape(1, -1), l1["Wr"])
        p2 = spmv(h, src_r, dst_r, zrows)
        h, st = _conv2(p2, cparts, h, l2["Wl"], l2["bl"].reshape(1, -1), l2["Wr"])
        if i < 3:
            h = _bn(h, st, bn["g"].reshape(1, -1), bn["b"].reshape(1, -1))
        else:
            w2t = jnp.broadcast_to(cls[2]["W"], (_D, _D // 2))
            b2t = jnp.broadcast_to(cls[2]["b"].reshape(1, 1), (1, _D))
            out = _final(h, st, bn["g"].reshape(1, -1), bn["b"].reshape(1, -1),
                         cls[0]["W"], cls[0]["b"].reshape(1, -1),
                         cls[1]["W"], cls[1]["b"].reshape(1, -1),
                         w2t, b2t)
    return out[:, 0:1]
